# baseline (speedup 1.0000x reference)
import functools

import jax
import jax.numpy as jnp
from jax import lax
from jax.experimental import pallas as pl
from jax.experimental.pallas import tpu as pltpu
from jax.experimental.pallas import tpu_sc as plsc

_NC = 2
_NS = 16
_L = 16
_NW = _NC * _NS

_BB = 128
_NB = 4


def _rsqrt(x):
    i = lax.bitcast_convert_type(x, jnp.int32)
    y = lax.bitcast_convert_type(jnp.int32(0x5F3759DF) - (i >> 1), jnp.float32)
    for _ in range(2):
        y = y * (1.5 - 0.5 * x * y * y)
    return y


def _tr_body(wt_ref, o_ref):
    x = wt_ref[...]
    eye = jnp.eye(x.shape[0], dtype=jnp.float32)
    y = lax.dot_general(x, eye, (((0,), (0,)), ((), ())),
                        precision=lax.Precision.HIGHEST,
                        preferred_element_type=jnp.float32)
    o_ref[...] = jnp.concatenate([y, y], axis=1)


def _row_major_table(W):
    WT = jnp.transpose(W)
    D, V = WT.shape
    BV = 1024
    out = pl.pallas_call(
        _tr_body,
        grid=(pl.cdiv(V, BV),),
        in_specs=[pl.BlockSpec((D, BV), lambda i: (0, i))],
        out_specs=pl.BlockSpec((BV, 2 * D), lambda i: (i, 0)),
        out_shape=jax.ShapeDtypeStruct((V, 2 * D), jnp.float32),
    )(WT)
    return out.reshape(2 * V, D)


def _ot_body(x_ref, o_ref):
    x2 = x_ref[0]
    d = o_ref.shape[1]
    halves = []
    for p in range(2):
        eyep = jnp.eye(d, 2 * d, k=d * p, dtype=jnp.float32)
        halves.append(lax.dot_general(eyep, x2, (((1,), (1,)), ((), ())),
                                      precision=lax.Precision.HIGHEST,
                                      preferred_element_type=jnp.float32))
    o_ref[0] = jnp.concatenate(halves, axis=1)


def _packed_to_hdb(y2, D, B):
    H = y2.shape[0]
    return pl.pallas_call(
        _ot_body,
        grid=(H, B // 1024),
        in_specs=[pl.BlockSpec((1, 512, 128), lambda h, k: (h, k, 0))],
        out_specs=pl.BlockSpec((1, D, 1024), lambda h, k: (h, 0, k)),
        out_shape=jax.ShapeDtypeStruct((H, D, B), jnp.float32),
    )(y2)


def kernel(inp, W):
    B, H = inp.shape
    V, D = W.shape
    blk_w = B // (_NW * _BB)
    n_units = blk_w * H
    rounds = n_units // _NB
    n_v = D // _L

    idxT = jnp.transpose(inp * 2)

    mesh = plsc.VectorSubcoreMesh(
        core_axis_name="c", subcore_axis_name="s",
        num_cores=_NC, num_subcores=_NS)

    @functools.partial(
        pl.kernel,
        out_type=jax.ShapeDtypeStruct((H, B * D // 128, 128), jnp.float32),
        mesh=mesh,
        scratch_types=(
            [pltpu.VMEM((_NB, _BB), jnp.int32),
             pltpu.VMEM((_NB, _BB, D), jnp.float32)]
            + [pltpu.SemaphoreType.DMA] * (2 * _NB)
        ),
        compiler_params=pltpu.CompilerParams(use_tc_tiling_on_sc=False),
    )
    def _emb(idx_hbm, table_hbm, out_hbm, idx_v, rows_v, *sems):
        gsem = sems[:_NB]
        wsem = sems[_NB:]
        wid = lax.axis_index("s") * _NC + lax.axis_index("c")
        lanes = lax.iota(jnp.int32, _L)

        def unit_hb(ci):
            blk = ci // H
            h = ci - blk * H
            return h, wid * (blk_w * _BB) + blk * _BB

        def fire_gather(ci, b):
            h, b0 = unit_hb(ci)
            pltpu.sync_copy(idx_hbm.at[h, pl.ds(b0, _BB)], idx_v.at[b])
            pltpu.async_copy(table_hbm.at[idx_v.at[b]], rows_v.at[b], gsem[b])

        def drain_gather(b):
            pltpu.make_async_copy(table_hbm.at[pl.ds(0, _BB)], rows_v.at[b],
                                  gsem[b]).wait()

        def fire_wb(ci, b):
            h, b0 = unit_hb(ci)
            k = b0 // 1024
            rem = b0 - k * 1024
            p = rem // 512
            q0 = rem - p * 512
            pltpu.async_copy(
                rows_v.at[b],
                out_hbm.at[h, pl.ds(k * 512 + q0, _BB), pl.ds(D * p, D)],
                wsem[b])

        def drain_wb(b):
            pltpu.make_async_copy(rows_v.at[b],
                                  out_hbm.at[0, pl.ds(0, _BB), pl.ds(0, D)],
                                  wsem[b]).wait()

        def compute(b):
            @plsc.parallel_loop(0, _BB, unroll=4)
            def _(l):
                v = [rows_v[b, l, pl.ds(j * _L, _L)] for j in range(n_v)]
                ss = v[0] * v[0]
                for vv in v[1:]:
                    ss = ss + vv * vv
                for k in (8, 4, 2, 1):
                    ss = ss + ss.at[lanes ^ k].get(mode="promise_in_bounds")
                rs = _rsqrt(jnp.maximum(ss, 1e-24))
                for j in range(n_v):
                    rows_v[b, l, pl.ds(j * _L, _L)] = v[j] * rs

        fire_gather(0, 0)
        fire_gather(1, 1)

        def round_body(r, _):
            for b in range(_NB):
                ci = _NB * r + b
                drain_gather(b)
                compute(b)
                fire_wb(ci, b)
                nxt = ci + 2
                nb = (b + 2) % _NB
                if b < 2:
                    @pl.when(r > 0)
                    def _():
                        drain_wb(nb)
                    fire_gather(nxt, nb)
                else:
                    @pl.when(r < rounds - 1)
                    def _():
                        drain_wb(nb)
                        fire_gather(nxt, nb)
            return 0

        lax.fori_loop(0, rounds, round_body, 0)
        for b in range(_NB):
            drain_wb(b)

    out_packed = _emb(idxT, _row_major_table(W))
    out_hdb = _packed_to_hdb(out_packed, D, B)
    return jnp.transpose(out_hdb, (2, 0, 1))


# speedup vs baseline: 1.0162x; 1.0162x over previous
import functools

import jax
import jax.numpy as jnp
from jax import lax
from jax.experimental import pallas as pl
from jax.experimental.pallas import tpu as pltpu
from jax.experimental.pallas import tpu_sc as plsc

_NC = 2
_NS = 16
_L = 16
_NW = _NC * _NS

_BB = 128
_NB = 4


def _rsqrt(x):
    i = lax.bitcast_convert_type(x, jnp.int32)
    y = lax.bitcast_convert_type(jnp.int32(0x5F3759DF) - (i >> 1), jnp.float32)
    for _ in range(2):
        y = y * (1.5 - 0.5 * x * y * y)
    return y


def kernel(inp, W):
    B, H = inp.shape
    V, D = W.shape
    blk_w = B // (_NW * _BB)
    n_units = blk_w * H
    rounds = n_units // _NB
    n_v = D // _L

    idxT = jnp.transpose(inp)

    mesh = plsc.VectorSubcoreMesh(
        core_axis_name="c", subcore_axis_name="s",
        num_cores=_NC, num_subcores=_NS)

    @functools.partial(
        pl.kernel,
        out_type=jax.ShapeDtypeStruct((H, D, B), jnp.float32),
        mesh=mesh,
        scratch_types=(
            [pltpu.VMEM((_NB, _BB), jnp.int32),
             pltpu.VMEM((_NB, _BB, D), jnp.float32),
             pltpu.VMEM((_NB, D, _BB), jnp.float32)]
            + [pltpu.SemaphoreType.DMA] * (2 * _NB)
        ),
        compiler_params=pltpu.CompilerParams(
            use_tc_tiling_on_sc=False, needs_layout_passes=False),
    )
    def _emb(idx_hbm, table_hbm, out_hbm, idx_v, rows_v, t_v, *sems):
        gsem = sems[:_NB]
        wsem = sems[_NB:]
        wid = lax.axis_index("s") * _NC + lax.axis_index("c")
        lanes = lax.iota(jnp.int32, _L)

        def unit_hb(ci):
            blk = ci // H
            h = ci - blk * H
            return h, wid * (blk_w * _BB) + blk * _BB

        def fire_gather(ci, b):
            h, b0 = unit_hb(ci)
            pltpu.sync_copy(idx_hbm.at[h, pl.ds(b0, _BB)], idx_v.at[b])
            pltpu.async_copy(table_hbm.at[idx_v.at[b]], rows_v.at[b], gsem[b])

        def drain_gather(b):
            pltpu.make_async_copy(table_hbm.at[pl.ds(0, _BB)], rows_v.at[b],
                                  gsem[b]).wait()

        def fire_wb(ci, b):
            h, b0 = unit_hb(ci)
            pltpu.async_copy(t_v.at[b], out_hbm.at[h, :, pl.ds(b0, _BB)],
                             wsem[b])

        def drain_wb(b):
            pltpu.make_async_copy(t_v.at[b], out_hbm.at[0, :, pl.ds(0, _BB)],
                                  wsem[b]).wait()

        def compute(b):
            rowi = [jnp.int32(j * _L) + lanes for j in range(n_v)]
            bi = jnp.full((_L,), b, jnp.int32)

            @plsc.parallel_loop(0, _BB, unroll=4)
            def _(l):
                v = [rows_v[b, l, pl.ds(j * _L, _L)] for j in range(n_v)]
                ss = v[0] * v[0]
                for vv in v[1:]:
                    ss = ss + vv * vv
                for k in (8, 4, 2, 1):
                    ss = ss + ss.at[lanes ^ k].get(mode="promise_in_bounds")
                rs = _rsqrt(jnp.maximum(ss, 1e-24))
                coli = jnp.full((_L,), l, jnp.int32)
                for j in range(n_v):
                    plsc.store_scatter(t_v, [bi, rowi[j], coli], v[j] * rs)

        fire_gather(0, 0)
        fire_gather(1, 1)

        def round_body(r, _):
            for b in range(_NB):
                ci = _NB * r + b
                drain_gather(b)

                @pl.when(r > 0)
                def _():
                    drain_wb(b)

                compute(b)
                fire_wb(ci, b)
                nxt = ci + 2
                nb = (b + 2) % _NB
                if b < 2:
                    fire_gather(nxt, nb)
                else:
                    @pl.when(r < rounds - 1)
                    def _():
                        fire_gather(nxt, nb)
            return 0

        lax.fori_loop(0, rounds, round_body, 0)
        for b in range(_NB):
            drain_wb(b)

    out_hdb = _emb(idxT, W)
    return jnp.transpose(out_hdb, (2, 0, 1))


# speedup vs baseline: 1.4984x; 1.4745x over previous
import functools

import jax
import jax.numpy as jnp
from jax import lax
from jax.experimental import pallas as pl
from jax.experimental.pallas import tpu as pltpu
from jax.experimental.pallas import tpu_sc as plsc

_NC = 2
_NS = 16
_L = 16
_NW = _NC * _NS

_C = 256
_G = 128
_NB = 4


def _rsqrt(x):
    i = lax.bitcast_convert_type(x, jnp.int32)
    y = lax.bitcast_convert_type(jnp.int32(0x5F3759DF) - (i >> 1), jnp.float32)
    for _ in range(2):
        y = y * (1.5 - 0.5 * x * y * y)
    return y


def kernel(inp, W):
    B, H = inp.shape
    V, D = W.shape
    N = B * H
    per_w = N // _NW
    n_chunks = per_w // _C
    rounds = n_chunks // _NB
    n_g = _C // _G
    n_v = D // _L

    idx2d = inp.reshape(N // _G, _G)

    mesh = plsc.VectorSubcoreMesh(
        core_axis_name="c", subcore_axis_name="s",
        num_cores=_NC, num_subcores=_NS)

    @functools.partial(
        pl.kernel,
        out_type=jax.ShapeDtypeStruct((N, D), jnp.float32),
        mesh=mesh,
        scratch_types=(
            [pltpu.VMEM((_NB, n_g, _G), jnp.int32),
             pltpu.VMEM((_NB, _C, D), jnp.float32)]
            + [pltpu.SemaphoreType.DMA] * (2 * _NB)
        ),
        compiler_params=pltpu.CompilerParams(use_tc_tiling_on_sc=False),
    )
    def _emb(idx_hbm, table_hbm, out_hbm, idx_v, rows_v, *sems):
        gsem = sems[:_NB]
        wsem = sems[_NB:]
        wid = lax.axis_index("s") * _NC + lax.axis_index("c")
        lanes = lax.iota(jnp.int32, _L)

        def fire_gather(ci, b):
            pltpu.sync_copy(
                idx_hbm.at[pl.ds(wid * (per_w // _G) + ci * n_g, n_g)],
                idx_v.at[b])
            for j in range(n_g):
                pltpu.async_copy(table_hbm.at[idx_v.at[b, j]],
                                 rows_v.at[b, pl.ds(j * _G, _G)], gsem[b])

        def drain_gather(b):
            pltpu.make_async_copy(out_hbm.at[pl.ds(0, _C)], rows_v.at[b],
                                  gsem[b]).wait()

        def fire_wb(ci, b):
            pltpu.async_copy(rows_v.at[b],
                             out_hbm.at[pl.ds(wid * per_w + ci * _C, _C)],
                             wsem[b])

        def drain_wb(b):
            pltpu.make_async_copy(rows_v.at[b], out_hbm.at[pl.ds(0, _C)],
                                  wsem[b]).wait()

        def compute(b):
            @plsc.parallel_loop(0, _C, unroll=8)
            def _(r):
                v = [rows_v[b, r, pl.ds(j * _L, _L)] for j in range(n_v)]
                ss = v[0] * v[0]
                for vv in v[1:]:
                    ss = ss + vv * vv
                for k in (8, 4, 2, 1):
                    ss = ss + ss.at[lanes ^ k].get(mode="promise_in_bounds")
                rs = _rsqrt(jnp.maximum(ss, 1e-24))
                for j in range(n_v):
                    rows_v[b, r, pl.ds(j * _L, _L)] = v[j] * rs

        fire_gather(0, 0)
        fire_gather(1, 1)

        def round_body(r, _):
            for b in range(_NB):
                ci = _NB * r + b
                drain_gather(b)
                compute(b)
                fire_wb(ci, b)
                jb = (b + 2) % _NB
                cj = ci + 2
                if b < 2:
                    @pl.when(r > 0)
                    def _():
                        drain_wb(jb)
                    fire_gather(cj, jb)
                else:
                    @pl.when(r < rounds - 1)
                    def _():
                        drain_wb(jb)
                        fire_gather(cj, jb)
            return 0

        lax.fori_loop(0, rounds, round_body, 0)
        for b in range(_NB):
            drain_wb(b)

    out = _emb(idx2d, W)
    return out.reshape(B, H, D)
